# baseline (speedup 1.0000x reference)
import jax
import jax.numpy as jnp
from jax import lax
from jax.experimental import pallas as pl
from jax.experimental.pallas import tpu as pltpu
from jax.experimental.pallas import tpu_sc as plsc

NC = 2
NS = 16
NW = NC * NS
S = 200
BB = 128


def _gather_body(idxT_hbm, table_hbm, out_hbm, idx_v, rows_v, tout_v, gsem):
    wid = lax.axis_index("s") * NC + lax.axis_index("c")
    pltpu.sync_copy(idxT_hbm.at[:, pl.ds(wid * BB, BB)], idx_v)
    iota = lax.iota(jnp.int32, 16)

    def unit(s, carry):
        pltpu.async_copy(table_hbm.at[idx_v.at[s]], rows_v, gsem).wait()

        def blk(bk, c2):
            rid = iota + bk * 16
            for fg in range(4):
                for fr in range(8):
                    vec = plsc.load_gather(
                        rows_v, [rid, jnp.full((16,), fg * 8 + fr, jnp.int32)])
                    tout_v[fg, pl.ds(fr * BB + bk * 16, 16)] = vec
            return c2

        lax.fori_loop(0, 8, blk, 0)
        pltpu.sync_copy(tout_v, out_hbm.at[s, :, pl.ds(wid * 1024, 1024)])
        return carry

    lax.fori_loop(0, S, unit, 0)


def kernel(inputs, embedding):
    bt, s = inputs.shape
    v, d = embedding.shape
    assert (bt, s, d) == (NW * BB, S, 32)

    idxT = inputs.T.astype(jnp.int32)
    mesh = plsc.VectorSubcoreMesh(core_axis_name="c", subcore_axis_name="s")
    k = pl.kernel(
        _gather_body,
        out_type=jax.ShapeDtypeStruct((S, 4, 8 * BB * NW), jnp.float32),
        mesh=mesh,
        scratch_types=[
            pltpu.VMEM((S, BB), jnp.int32),
            pltpu.VMEM((BB, d), jnp.float32),
            pltpu.VMEM((4, 8 * BB), jnp.float32),
            pltpu.SemaphoreType.DMA,
        ],
        compiler_params=pltpu.CompilerParams(
            use_tc_tiling_on_sc=False, needs_layout_passes=False),
    )
    out5 = k(idxT, embedding)
    return (out5.reshape(S, 4, NW, 8, BB)
                .transpose(2, 4, 0, 1, 3)
                .reshape(bt, s, d))


# speedup vs baseline: 1.1023x; 1.1023x over previous
import jax
import jax.numpy as jnp
from jax import lax
from jax.experimental import pallas as pl
from jax.experimental.pallas import tpu as pltpu
from jax.experimental.pallas import tpu_sc as plsc

NC = 2
NS = 16
NW = NC * NS
S = 200
BB = 128


def _gather_body(idxT_hbm, table_hbm, out_hbm, idx_v, rows_v, tout_v,
                 gsem0, gsem1, ssem0, ssem1):
    wid = lax.axis_index("s") * NC + lax.axis_index("c")
    pltpu.sync_copy(idxT_hbm.at[:, pl.ds(wid * BB, BB)], idx_v)
    iota = lax.iota(jnp.int32, 16)

    def gather(s, p, sem):
        return pltpu.make_async_copy(
            table_hbm.at[idx_v.at[s]], rows_v.at[p], sem)

    def store(s, p, sem):
        return pltpu.make_async_copy(
            tout_v.at[p], out_hbm.at[s, :, pl.ds(wid * 1024, 1024)], sem)

    def transpose(p):
        for bk in range(8):
            rid = iota + bk * 16
            for fg in range(4):
                for fr in range(8):
                    vec = plsc.load_gather(
                        rows_v.at[p],
                        [rid, jnp.full((16,), fg * 8 + fr, jnp.int32)])
                    tout_v[p, fg, pl.ds(fr * BB + bk * 16, 16)] = vec

    gather(0, 0, gsem0).start()
    gather(1, 1, gsem1).start()

    def pair(j2, carry):
        for p, gsem, ssem in ((0, gsem0, ssem0), (1, gsem1, ssem1)):
            s = 2 * j2 + p
            gather(s, p, gsem).wait()

            @pl.when(j2 >= 1)
            def _():
                store(s - 2, p, ssem).wait()

            transpose(p)

            @pl.when(j2 < S // 2 - 1)
            def _():
                gather(s + 2, p, gsem).start()

            store(s, p, ssem).start()
        return carry

    lax.fori_loop(0, S // 2, pair, 0)
    store(S - 2, 0, ssem0).wait()
    store(S - 1, 1, ssem1).wait()


def kernel(inputs, embedding):
    bt, s = inputs.shape
    v, d = embedding.shape
    assert (bt, s, d) == (NW * BB, S, 32)

    idxT = inputs.T.astype(jnp.int32)
    mesh = plsc.VectorSubcoreMesh(core_axis_name="c", subcore_axis_name="s")
    k = pl.kernel(
        _gather_body,
        out_type=jax.ShapeDtypeStruct((S, 4, 8 * BB * NW), jnp.float32),
        mesh=mesh,
        scratch_types=[
            pltpu.VMEM((S, BB), jnp.int32),
            pltpu.VMEM((2, BB, d), jnp.float32),
            pltpu.VMEM((2, 4, 8 * BB), jnp.float32),
            pltpu.SemaphoreType.DMA,
            pltpu.SemaphoreType.DMA,
            pltpu.SemaphoreType.DMA,
            pltpu.SemaphoreType.DMA,
        ],
        compiler_params=pltpu.CompilerParams(
            use_tc_tiling_on_sc=False, needs_layout_passes=False),
    )
    out5 = k(idxT, embedding)
    return (out5.reshape(S, 4, NW, 8, BB)
                .transpose(2, 4, 0, 1, 3)
                .reshape(bt, s, d))


# speedup vs baseline: 1.5204x; 1.3793x over previous
import jax
import jax.numpy as jnp
from jax import lax
from jax.experimental import pallas as pl
from jax.experimental.pallas import tpu as pltpu
from jax.experimental.pallas import tpu_sc as plsc

NC = 2
NS = 16
NW = NC * NS
S = 200
BB = 128


def _gather_body(idxT_hbm, table_hbm, out_hbm, idx_v, rows_v, tout_v,
                 gsem0, gsem1, ssem0, ssem1):
    wid = lax.axis_index("s") * NC + lax.axis_index("c")
    pltpu.sync_copy(idxT_hbm.at[:, pl.ds(wid * BB, BB)], idx_v)
    iota = lax.iota(jnp.int32, 16)

    def gather(s, p, sem):
        return pltpu.make_async_copy(
            table_hbm.at[idx_v.at[s]], rows_v.at[p], sem)

    def store(s, p, sem):
        return pltpu.make_async_copy(
            tout_v.at[p], out_hbm.at[s, :, pl.ds(wid * 1024, 1024)], sem)

    def transpose(p):
        for bk in range(8):
            rid = iota + bk * 16
            for fg in range(4):
                vecs = [
                    plsc.load_gather(
                        rows_v.at[p],
                        [rid, jnp.full((16,), fg * 8 + fr, jnp.int32)])
                    for fr in range(8)
                ]
                for fr in range(8):
                    tout_v[p, fg, pl.ds(fr * BB + bk * 16, 16)] = vecs[fr]

    gather(0, 0, gsem0).start()
    gather(1, 1, gsem1).start()

    def pair(j2, carry):
        for p, gsem, ssem in ((0, gsem0, ssem0), (1, gsem1, ssem1)):
            s = 2 * j2 + p
            gather(s, p, gsem).wait()

            @pl.when(j2 >= 1)
            def _():
                store(s - 2, p, ssem).wait()

            transpose(p)

            @pl.when(j2 < S // 2 - 1)
            def _():
                gather(s + 2, p, gsem).start()

            store(s, p, ssem).start()
        return carry

    lax.fori_loop(0, S // 2, pair, 0)
    store(S - 2, 0, ssem0).wait()
    store(S - 1, 1, ssem1).wait()


def kernel(inputs, embedding):
    bt, s = inputs.shape
    v, d = embedding.shape
    assert (bt, s, d) == (NW * BB, S, 32)

    idxT = inputs.T.astype(jnp.int32)
    mesh = plsc.VectorSubcoreMesh(core_axis_name="c", subcore_axis_name="s")
    k = pl.kernel(
        _gather_body,
        out_type=jax.ShapeDtypeStruct((S, 4, 8 * BB * NW), jnp.float32),
        mesh=mesh,
        scratch_types=[
            pltpu.VMEM((S, BB), jnp.int32),
            pltpu.VMEM((2, BB, d), jnp.float32),
            pltpu.VMEM((2, 4, 8 * BB), jnp.float32),
            pltpu.SemaphoreType.DMA,
            pltpu.SemaphoreType.DMA,
            pltpu.SemaphoreType.DMA,
            pltpu.SemaphoreType.DMA,
        ],
        compiler_params=pltpu.CompilerParams(
            use_tc_tiling_on_sc=False, needs_layout_passes=False),
    )
    out5 = k(idxT, embedding)
    return (out5.reshape(S, 4, NW, 8, BB)
                .transpose(2, 4, 0, 1, 3)
                .reshape(bt, s, d))
